# baseline (speedup 1.0000x reference)
import functools

import jax
import jax.numpy as jnp
from jax import lax
from jax.experimental import pallas as pl
from jax.experimental.pallas import tpu as pltpu
from jax.experimental.pallas import tpu_sc as plsc

F32 = jnp.float32
N = 10000
E = 320000
D = 128
G = 64
NC, NS = 2, 16
NW = NC * NS
EPW = E // NW
CH = 80
NCH = EPW // CH
RST = 400
NRC = N // RST
BLK = 1000
GRID = N // BLK


@functools.cache
def _sc_kernels():
    mesh = plsc.VectorSubcoreMesh(core_axis_name="c", subcore_axis_name="s",
                                  num_cores=NC, num_subcores=NS)

    @functools.partial(
        pl.kernel,
        mesh=mesh,
        out_type=jax.ShapeDtypeStruct((NW, N), F32),
        scratch_types=[
            pltpu.VMEM((EPW // 16, 16), jnp.int32),
            pltpu.VMEM((N,), F32),
        ],
        compiler_params=pltpu.CompilerParams(needs_layout_passes=False),
    )
    def _sc_deg(dst_hbm, out_hbm, idx_v, deg_v):
        cid = lax.axis_index("c")
        sid = lax.axis_index("s")
        wid = sid * NC + cid
        pltpu.sync_copy(dst_hbm.at[wid], idx_v)

        def zbody(i, c):
            deg_v[pl.ds(i * 16, 16)] = jnp.zeros((16,), F32)
            return c

        lax.fori_loop(0, N // 16, zbody, 0)
        ones = jnp.ones((16,), F32)

        def body(i, c):
            plsc.addupdate_scatter(deg_v, [idx_v[i]], ones)
            return c

        lax.fori_loop(0, EPW // 16, body, 0)
        pltpu.sync_copy(deg_v, out_hbm.at[wid])

    @functools.partial(
        pl.kernel,
        mesh=mesh,
        out_type=jax.ShapeDtypeStruct((NC, N, D), F32),
        scratch_types=[
            pltpu.VMEM((EPW,), jnp.int32),
            pltpu.VMEM((NCH, CH), jnp.int32),
            pltpu.VMEM((2, CH, D), F32),
            pltpu.VMEM_SHARED((N, D), F32),
            pltpu.SemaphoreType.DMA,
            pltpu.SemaphoreType.DMA,
            pltpu.SemaphoreType.DMA,
            pltpu.SemaphoreType.DMA,
        ],
    )
    def _sc_scatter(hs_hbm, src_hbm, dst_hbm, out_hbm,
                    src_v, dst_v, rows_v, acc_sh, sem0, sem1, sem2, sem3):
        cid = lax.axis_index("c")
        sid = lax.axis_index("s")
        wid = sid * NC + cid
        pltpu.sync_copy(src_hbm.at[wid], src_v)
        pltpu.sync_copy(dst_hbm.at[wid], dst_v)
        for r in range((NRC + NS - 1) // NS):
            chunk = r * NS + sid

            @pl.when(chunk < NRC)
            def _():
                base = pl.multiple_of(chunk * RST, 8)
                pltpu.sync_copy(hs_hbm.at[pl.ds(base, RST)],
                                acc_sh.at[pl.ds(base, RST)])
        plsc.subcore_barrier()

        def gidx(j):
            return src_v.at[pl.ds(j * CH, CH)]

        gsem = (sem0, sem1)
        ssem = (sem2, sem3)
        pltpu.async_copy(hs_hbm.at[gidx(0)], rows_v.at[0], gsem[0])
        pltpu.async_copy(hs_hbm.at[gidx(1)], rows_v.at[1], gsem[1])

        def step(j, b):
            pltpu.make_async_copy(hs_hbm.at[gidx(j)],
                                  rows_v.at[b], gsem[b]).wait()
            pltpu.async_copy(rows_v.at[b], acc_sh.at[dst_v.at[j]],
                             ssem[b], add=True)

            @pl.when(j >= 1)
            def _():
                pltpu.make_async_copy(rows_v.at[1 - b],
                                      acc_sh.at[dst_v.at[0]],
                                      ssem[1 - b]).wait()

                @pl.when(j + 1 < NCH)
                def _():
                    pltpu.async_copy(hs_hbm.at[gidx(j + 1)],
                                     rows_v.at[1 - b], gsem[1 - b])

        def body(t, c):
            for b in range(2):
                step(t * 2 + b, b)
            return c

        lax.fori_loop(0, NCH // 2, body, 0)
        if NCH % 2:
            step(NCH - 1, (NCH - 1) % 2)
        pltpu.make_async_copy(rows_v.at[(NCH - 1) % 2],
                              acc_sh.at[dst_v.at[0]],
                              ssem[(NCH - 1) % 2]).wait()
        plsc.subcore_barrier()
        for r in range((NRC + NS - 1) // NS):
            chunk = r * NS + sid

            @pl.when(chunk < NRC)
            def _():
                base = pl.multiple_of(chunk * RST, 8)
                pltpu.sync_copy(acc_sh.at[pl.ds(base, RST)],
                                out_hbm.at[cid, pl.ds(base, RST)])

    PCH = 80
    NPC = N // PCH

    @functools.partial(
        pl.kernel,
        mesh=mesh,
        out_type=jax.ShapeDtypeStruct((NW, G, D), F32),
        scratch_types=[
            pltpu.VMEM((PCH, D), F32),
            pltpu.VMEM((PCH, D), F32),
            pltpu.VMEM((PCH, D), F32),
            pltpu.VMEM((PCH,), F32),
            pltpu.VMEM((PCH,), jnp.int32),
            pltpu.VMEM((D,), F32),
            pltpu.VMEM((G + 8, D), F32),
        ],
    )
    def _sc_pool(a_hbm, hs_hbm, dis_hbm, batch_hbm, b3_hbm, out_hbm,
                 a0_v, a1_v, hs_v, dis_v, bat_v, b3_v, out_v):
        cid = lax.axis_index("c")
        sid = lax.axis_index("s")
        wid = sid * NC + cid
        pltpu.sync_copy(b3_hbm, b3_v)

        def ibody(g, c):
            for k in range(8):
                out_v[g, pl.ds(k * 16, 16)] = jnp.full((16,), -jnp.inf, F32)
            return c

        lax.fori_loop(0, G + 8, ibody, 0)

        def do_chunk(chunk):
            base = pl.multiple_of(chunk * PCH, 8)
            pltpu.sync_copy(a_hbm.at[0, pl.ds(base, PCH)], a0_v)
            pltpu.sync_copy(a_hbm.at[1, pl.ds(base, PCH)], a1_v)
            pltpu.sync_copy(hs_hbm.at[pl.ds(base, PCH)], hs_v)
            pltpu.sync_copy(dis_hbm.at[pl.ds(base, PCH)], dis_v)
            pltpu.sync_copy(batch_hbm.at[pl.ds(base, PCH)], bat_v)

            def flush(prev, accs):
                for k in range(8):
                    sl = pl.ds(k * 16, 16)
                    out_v[prev, sl] = jnp.maximum(out_v[prev, sl], accs[k])

            def gbody(g, carry):
                prev = carry[0]
                accs = list(carry[1:])
                bat16 = bat_v[pl.ds(g * 16, 16)]
                dis16 = dis_v[pl.ds(g * 16, 16)]
                for l in range(16):
                    i = g * 16 + l
                    gid = bat16[l]
                    dv = dis16[l]
                    changed = gid != prev

                    @pl.when(changed)
                    def _():
                        flush(prev, accs)

                    new_accs = []
                    for k in range(8):
                        sl = pl.ds(k * 16, 16)
                        h = a0_v[i, sl] + a1_v[i, sl] - hs_v[i, sl]
                        h = jnp.maximum(dv * h + b3_v[sl], 0.0)
                        new_accs.append(
                            jnp.where(changed, h, jnp.maximum(accs[k], h)))
                    accs = new_accs
                    prev = gid
                return (prev, *accs)

            ninf = jnp.full((16,), -jnp.inf, F32)
            carry0 = (jnp.int32(G), ninf, ninf, ninf, ninf,
                      ninf, ninf, ninf, ninf)
            fin = lax.fori_loop(0, PCH // 16, gbody, carry0)
            flush(fin[0], list(fin[1:]))

        for t in range((NPC + NW - 1) // NW):
            chunk = t * NW + wid

            @pl.when(chunk < NPC)
            def _():
                do_chunk(chunk)

        pltpu.sync_copy(out_v.at[pl.ds(0, G)], out_hbm.at[wid])

    return _sc_deg, _sc_scatter, _sc_pool


def _p0_body(degp_ref, x_ref, w_ref, dis_ref, hs_ref):
    ones = jnp.ones((NW, 1), F32)
    deg = lax.dot_general(degp_ref[...], ones, (((0,), (0,)), ((), ())),
                          preferred_element_type=F32)
    dis = lax.rsqrt(deg + 1.0)
    dis_ref[...] = dis
    hs_ref[...] = dis * jnp.dot(x_ref[...], w_ref[...],
                                preferred_element_type=F32)


def _tc_p0(x, w1, degp):
    return pl.pallas_call(
        _p0_body,
        out_shape=[
            jax.ShapeDtypeStruct((N, 1), F32),
            jax.ShapeDtypeStruct((N, D), F32),
        ],
    )(degp, x, w1)


def _c_body(a_ref, hs_ref, dis_ref, b_ref, w_ref, out_ref):
    s = a_ref[0] + a_ref[1] - hs_ref[...]
    h = jnp.maximum(dis_ref[...] * s + b_ref[...], 0.0)
    out_ref[...] = dis_ref[...] * jnp.dot(h, w_ref[...],
                                          preferred_element_type=F32)


def _tc_combine(a, hs, dis, b, w_next):
    return pl.pallas_call(
        _c_body,
        grid=(GRID,),
        in_specs=[
            pl.BlockSpec((NC, BLK, D), lambda i: (0, i, 0)),
            pl.BlockSpec((BLK, D), lambda i: (i, 0)),
            pl.BlockSpec((BLK, 1), lambda i: (i, 0)),
            pl.BlockSpec((1, D), lambda i: (0, 0)),
            pl.BlockSpec((D, D), lambda i: (0, 0)),
        ],
        out_specs=pl.BlockSpec((BLK, D), lambda i: (i, 0)),
        out_shape=jax.ShapeDtypeStruct((N, D), F32),
    )(a, hs, dis, b, w_next)


def _head_body(p_ref, lw1_ref, lb1_ref, lw2_ref, lb2_ref, out_ref):
    gmax = jnp.max(p_ref[...], axis=0)
    g2 = jnp.dot(gmax, lw1_ref[...],
                 preferred_element_type=F32) + lb1_ref[...]
    out_ref[...] = jnp.dot(g2, lw2_ref[...],
                           preferred_element_type=F32) + lb2_ref[...]


def _tc_head(partials, lw1, lb1, lw2, lb2):
    return pl.pallas_call(
        _head_body,
        out_shape=jax.ShapeDtypeStruct((G, 1), F32),
    )(partials, lw1, lb1, lw2, lb2)


def kernel(x, x2, edge_attr, edge_attr2, edge_index, batch,
           W1, b1, W2, b2, W3, b3, LW1, Lb1, LW2, Lb2):
    src3 = edge_index[0].reshape(NW, EPW)
    dst3 = edge_index[1].reshape(NW, NCH, CH)
    dst16 = edge_index[1].reshape(NW, EPW // 16, 16)
    b1r = b1.reshape(1, D)
    b2r = b2.reshape(1, D)
    lb1r = Lb1.reshape(1, D)
    lb2r = Lb2.reshape(1, 1)

    _sc_deg, _sc_scatter, _sc_pool = _sc_kernels()
    degp = _sc_deg(dst16)
    dis, hs1 = _tc_p0(x, W1, degp)
    a1 = _sc_scatter(hs1, src3, dst3)
    hs2 = _tc_combine(a1, hs1, dis, b1r, W2)
    a2 = _sc_scatter(hs2, src3, dst3)
    hs3 = _tc_combine(a2, hs2, dis, b2r, W3)
    a3 = _sc_scatter(hs3, src3, dst3)
    partials = _sc_pool(a3, hs3, dis.reshape(N), batch, b3)
    out = _tc_head(partials, LW1, lb1r, LW2, lb2r)
    return out


# speedup vs baseline: 1.2301x; 1.2301x over previous
import functools

import jax
import jax.numpy as jnp
from jax import lax
from jax.experimental import pallas as pl
from jax.experimental.pallas import tpu as pltpu
from jax.experimental.pallas import tpu_sc as plsc

F32 = jnp.float32
N = 10000
E = 320000
D = 128
G = 64
NC, NS = 2, 16
NW = NC * NS
EPW = E // NW
CH = 80
NCH = EPW // CH
RST = 400
NRC = N // RST
BLK = 1000
GRID = N // BLK


@functools.cache
def _sc_kernels():
    mesh = plsc.VectorSubcoreMesh(core_axis_name="c", subcore_axis_name="s",
                                  num_cores=NC, num_subcores=NS)

    @functools.partial(
        pl.kernel,
        mesh=mesh,
        out_type=jax.ShapeDtypeStruct((NW, N), F32),
        scratch_types=[
            pltpu.VMEM((EPW // 16, 16), jnp.int32),
            pltpu.VMEM((N,), F32),
        ],
        compiler_params=pltpu.CompilerParams(needs_layout_passes=False),
    )
    def _sc_deg(dst_hbm, out_hbm, idx_v, deg_v):
        cid = lax.axis_index("c")
        sid = lax.axis_index("s")
        wid = sid * NC + cid
        pltpu.sync_copy(dst_hbm.at[wid], idx_v)

        def zbody(i, c):
            deg_v[pl.ds(i * 16, 16)] = jnp.zeros((16,), F32)
            return c

        lax.fori_loop(0, N // 16, zbody, 0)
        ones = jnp.ones((16,), F32)

        def body(i, c):
            plsc.addupdate_scatter(deg_v, [idx_v[i]], ones)
            return c

        lax.fori_loop(0, EPW // 16, body, 0)
        pltpu.sync_copy(deg_v, out_hbm.at[wid])

    @functools.partial(
        pl.kernel,
        mesh=mesh,
        out_type=jax.ShapeDtypeStruct((NC, N, D), F32),
        scratch_types=[
            pltpu.VMEM((EPW,), jnp.int32),
            pltpu.VMEM((NCH, CH), jnp.int32),
            pltpu.VMEM((2, CH, D), F32),
            pltpu.VMEM_SHARED((N, D), F32),
            pltpu.SemaphoreType.DMA,
            pltpu.SemaphoreType.DMA,
            pltpu.SemaphoreType.DMA,
            pltpu.SemaphoreType.DMA,
        ],
    )
    def _sc_scatter(hs_hbm, src_hbm, dst_hbm, out_hbm,
                    src_v, dst_v, rows_v, acc_sh, sem0, sem1, sem2, sem3):
        cid = lax.axis_index("c")
        sid = lax.axis_index("s")
        wid = sid * NC + cid
        pltpu.sync_copy(src_hbm.at[wid], src_v)
        pltpu.sync_copy(dst_hbm.at[wid], dst_v)
        for r in range((NRC + NS - 1) // NS):
            chunk = r * NS + sid

            @pl.when(chunk < NRC)
            def _():
                base = pl.multiple_of(chunk * RST, 8)
                pltpu.sync_copy(hs_hbm.at[pl.ds(base, RST)],
                                acc_sh.at[pl.ds(base, RST)])
        plsc.subcore_barrier()

        def gidx(j):
            return src_v.at[pl.ds(j * CH, CH)]

        pltpu.async_copy(hs_hbm.at[gidx(0)], rows_v.at[0], sem0)
        pltpu.async_copy(hs_hbm.at[gidx(1)], rows_v.at[1], sem1)

        def step(j, b):
            sem = (sem0, sem1)[b]
            pltpu.make_async_copy(hs_hbm.at[gidx(j)],
                                  rows_v.at[b], sem).wait()
            pltpu.sync_copy(rows_v.at[b], acc_sh.at[dst_v.at[j]], add=True)

            @pl.when(j + 2 < NCH)
            def _():
                pltpu.async_copy(hs_hbm.at[gidx(j + 2)], rows_v.at[b], sem)

        def body(t, c):
            for b in range(2):
                step(t * 2 + b, b)
            return c

        lax.fori_loop(0, NCH // 2, body, 0)
        if NCH % 2:
            step(NCH - 1, (NCH - 1) % 2)
        plsc.subcore_barrier()
        for r in range((NRC + NS - 1) // NS):
            chunk = r * NS + sid

            @pl.when(chunk < NRC)
            def _():
                base = pl.multiple_of(chunk * RST, 8)
                pltpu.sync_copy(acc_sh.at[pl.ds(base, RST)],
                                out_hbm.at[cid, pl.ds(base, RST)])

    PCH = 80
    NPC = N // PCH

    @functools.partial(
        pl.kernel,
        mesh=mesh,
        out_type=jax.ShapeDtypeStruct((NW, G, D), F32),
        scratch_types=[
            pltpu.VMEM((PCH, D), F32),
            pltpu.VMEM((PCH, D), F32),
            pltpu.VMEM((PCH, D), F32),
            pltpu.VMEM((PCH,), F32),
            pltpu.VMEM((PCH,), jnp.int32),
            pltpu.VMEM((D,), F32),
            pltpu.VMEM((G + 8, D), F32),
        ],
    )
    def _sc_pool(a_hbm, hs_hbm, dis_hbm, batch_hbm, b3_hbm, out_hbm,
                 a0_v, a1_v, hs_v, dis_v, bat_v, b3_v, out_v):
        cid = lax.axis_index("c")
        sid = lax.axis_index("s")
        wid = sid * NC + cid
        pltpu.sync_copy(b3_hbm, b3_v)

        def ibody(g, c):
            for k in range(8):
                out_v[g, pl.ds(k * 16, 16)] = jnp.full((16,), -jnp.inf, F32)
            return c

        lax.fori_loop(0, G + 8, ibody, 0)

        def do_chunk(chunk):
            base = pl.multiple_of(chunk * PCH, 8)
            pltpu.sync_copy(a_hbm.at[0, pl.ds(base, PCH)], a0_v)
            pltpu.sync_copy(a_hbm.at[1, pl.ds(base, PCH)], a1_v)
            pltpu.sync_copy(hs_hbm.at[pl.ds(base, PCH)], hs_v)
            pltpu.sync_copy(dis_hbm.at[pl.ds(base, PCH)], dis_v)
            pltpu.sync_copy(batch_hbm.at[pl.ds(base, PCH)], bat_v)

            def flush(prev, accs):
                for k in range(8):
                    sl = pl.ds(k * 16, 16)
                    out_v[prev, sl] = jnp.maximum(out_v[prev, sl], accs[k])

            def gbody(g, carry):
                prev = carry[0]
                accs = list(carry[1:])
                bat16 = bat_v[pl.ds(g * 16, 16)]
                dis16 = dis_v[pl.ds(g * 16, 16)]
                for l in range(16):
                    i = g * 16 + l
                    gid = bat16[l]
                    dv = dis16[l]
                    changed = gid != prev

                    @pl.when(changed)
                    def _():
                        flush(prev, accs)

                    new_accs = []
                    for k in range(8):
                        sl = pl.ds(k * 16, 16)
                        h = a0_v[i, sl] + a1_v[i, sl] - hs_v[i, sl]
                        h = jnp.maximum(dv * h + b3_v[sl], 0.0)
                        new_accs.append(
                            jnp.where(changed, h, jnp.maximum(accs[k], h)))
                    accs = new_accs
                    prev = gid
                return (prev, *accs)

            ninf = jnp.full((16,), -jnp.inf, F32)
            carry0 = (jnp.int32(G), ninf, ninf, ninf, ninf,
                      ninf, ninf, ninf, ninf)
            fin = lax.fori_loop(0, PCH // 16, gbody, carry0)
            flush(fin[0], list(fin[1:]))

        for t in range((NPC + NW - 1) // NW):
            chunk = t * NW + wid

            @pl.when(chunk < NPC)
            def _():
                do_chunk(chunk)

        pltpu.sync_copy(out_v.at[pl.ds(0, G)], out_hbm.at[wid])

    return _sc_deg, _sc_scatter, _sc_pool


def _p0_body(degp_ref, x_ref, w_ref, dis_ref, hs_ref):
    ones = jnp.ones((NW, 1), F32)
    deg = lax.dot_general(degp_ref[...], ones, (((0,), (0,)), ((), ())),
                          preferred_element_type=F32)
    dis = lax.rsqrt(deg + 1.0)
    dis_ref[...] = dis
    hs_ref[...] = dis * jnp.dot(x_ref[...], w_ref[...],
                                preferred_element_type=F32)


def _tc_p0(x, w1, degp):
    return pl.pallas_call(
        _p0_body,
        out_shape=[
            jax.ShapeDtypeStruct((N, 1), F32),
            jax.ShapeDtypeStruct((N, D), F32),
        ],
    )(degp, x, w1)


def _c_body(a_ref, hs_ref, dis_ref, b_ref, w_ref, out_ref):
    s = a_ref[0] + a_ref[1] - hs_ref[...]
    h = jnp.maximum(dis_ref[...] * s + b_ref[...], 0.0)
    out_ref[...] = dis_ref[...] * jnp.dot(h, w_ref[...],
                                          preferred_element_type=F32)


def _tc_combine(a, hs, dis, b, w_next):
    return pl.pallas_call(
        _c_body,
        grid=(GRID,),
        in_specs=[
            pl.BlockSpec((NC, BLK, D), lambda i: (0, i, 0)),
            pl.BlockSpec((BLK, D), lambda i: (i, 0)),
            pl.BlockSpec((BLK, 1), lambda i: (i, 0)),
            pl.BlockSpec((1, D), lambda i: (0, 0)),
            pl.BlockSpec((D, D), lambda i: (0, 0)),
        ],
        out_specs=pl.BlockSpec((BLK, D), lambda i: (i, 0)),
        out_shape=jax.ShapeDtypeStruct((N, D), F32),
    )(a, hs, dis, b, w_next)


def _head_body(p_ref, lw1_ref, lb1_ref, lw2_ref, lb2_ref, out_ref):
    gmax = jnp.max(p_ref[...], axis=0)
    g2 = jnp.dot(gmax, lw1_ref[...],
                 preferred_element_type=F32) + lb1_ref[...]
    out_ref[...] = jnp.dot(g2, lw2_ref[...],
                           preferred_element_type=F32) + lb2_ref[...]


def _tc_head(partials, lw1, lb1, lw2, lb2):
    return pl.pallas_call(
        _head_body,
        out_shape=jax.ShapeDtypeStruct((G, 1), F32),
    )(partials, lw1, lb1, lw2, lb2)


def kernel(x, x2, edge_attr, edge_attr2, edge_index, batch,
           W1, b1, W2, b2, W3, b3, LW1, Lb1, LW2, Lb2):
    src3 = edge_index[0].reshape(NW, EPW)
    dst3 = edge_index[1].reshape(NW, NCH, CH)
    dst16 = edge_index[1].reshape(NW, EPW // 16, 16)
    b1r = b1.reshape(1, D)
    b2r = b2.reshape(1, D)
    lb1r = Lb1.reshape(1, D)
    lb2r = Lb2.reshape(1, 1)

    _sc_deg, _sc_scatter, _sc_pool = _sc_kernels()
    degp = _sc_deg(dst16)
    dis, hs1 = _tc_p0(x, W1, degp)
    a1 = _sc_scatter(hs1, src3, dst3)
    hs2 = _tc_combine(a1, hs1, dis, b1r, W2)
    a2 = _sc_scatter(hs2, src3, dst3)
    hs3 = _tc_combine(a2, hs2, dis, b2r, W3)
    a3 = _sc_scatter(hs3, src3, dst3)
    partials = _sc_pool(a3, hs3, dis.reshape(N), batch, b3)
    out = _tc_head(partials, LW1, lb1r, LW2, lb2r)
    return out


# speedup vs baseline: 1.2461x; 1.0129x over previous
import functools

import jax
import jax.numpy as jnp
from jax import lax
from jax.experimental import pallas as pl
from jax.experimental.pallas import tpu as pltpu
from jax.experimental.pallas import tpu_sc as plsc

F32 = jnp.float32
N = 10000
E = 320000
D = 128
G = 64
NC, NS = 2, 16
NW = NC * NS
EPW = E // NW
CH = 80
NCH = EPW // CH
RST = 400
NRC = N // RST
BLK = 1000
GRID = N // BLK


@functools.cache
def _sc_kernels():
    mesh = plsc.VectorSubcoreMesh(core_axis_name="c", subcore_axis_name="s",
                                  num_cores=NC, num_subcores=NS)

    @functools.partial(
        pl.kernel,
        mesh=mesh,
        out_type=jax.ShapeDtypeStruct((NW, N), F32),
        scratch_types=[
            pltpu.VMEM((EPW // 16, 16), jnp.int32),
            pltpu.VMEM((N,), F32),
        ],
        compiler_params=pltpu.CompilerParams(needs_layout_passes=False),
    )
    def _sc_deg(dst_hbm, out_hbm, idx_v, deg_v):
        cid = lax.axis_index("c")
        sid = lax.axis_index("s")
        wid = sid * NC + cid
        pltpu.sync_copy(dst_hbm.at[wid], idx_v)

        def zbody(i, c):
            deg_v[pl.ds(i * 16, 16)] = jnp.zeros((16,), F32)
            return c

        lax.fori_loop(0, N // 16, zbody, 0)
        ones = jnp.ones((16,), F32)

        def body(i, c):
            plsc.addupdate_scatter(deg_v, [idx_v[i]], ones)
            return c

        lax.fori_loop(0, EPW // 16, body, 0)
        pltpu.sync_copy(deg_v, out_hbm.at[wid])

    @functools.partial(
        pl.kernel,
        mesh=mesh,
        out_type=jax.ShapeDtypeStruct((NC, N, D), F32),
        scratch_types=[
            pltpu.VMEM((EPW,), jnp.int32),
            pltpu.VMEM((NCH, CH), jnp.int32),
            pltpu.VMEM((2, CH, D), F32),
            pltpu.VMEM_SHARED((N, D), F32),
            pltpu.SemaphoreType.DMA,
            pltpu.SemaphoreType.DMA,
            pltpu.SemaphoreType.DMA,
            pltpu.SemaphoreType.DMA,
        ],
    )
    def _sc_scatter(hs_hbm, src_hbm, dst_hbm, out_hbm,
                    src_v, dst_v, rows_v, acc_sh, sem0, sem1, sem2, sem3):
        cid = lax.axis_index("c")
        sid = lax.axis_index("s")
        wid = sid * NC + cid
        pltpu.async_copy(src_hbm.at[wid], src_v, sem0)
        pltpu.async_copy(dst_hbm.at[wid], dst_v, sem1)
        isems = (sem2, sem3)
        for r in range((NRC + NS - 1) // NS):
            chunk = r * NS + sid

            @pl.when(chunk < NRC)
            def _():
                base = pl.multiple_of(chunk * RST, 8)
                pltpu.async_copy(hs_hbm.at[pl.ds(base, RST)],
                                 acc_sh.at[pl.ds(base, RST)], isems[r])
        pltpu.make_async_copy(src_hbm.at[wid], src_v, sem0).wait()
        pltpu.make_async_copy(dst_hbm.at[wid], dst_v, sem1).wait()
        for r in range((NRC + NS - 1) // NS):
            chunk = r * NS + sid

            @pl.when(chunk < NRC)
            def _():
                base = pl.multiple_of(chunk * RST, 8)
                pltpu.make_async_copy(hs_hbm.at[pl.ds(base, RST)],
                                      acc_sh.at[pl.ds(base, RST)],
                                      isems[r]).wait()
        plsc.subcore_barrier()

        def gidx(j):
            return src_v.at[pl.ds(j * CH, CH)]

        pltpu.async_copy(hs_hbm.at[gidx(0)], rows_v.at[0], sem0)
        pltpu.async_copy(hs_hbm.at[gidx(1)], rows_v.at[1], sem1)

        def step(j, b):
            sem = (sem0, sem1)[b]
            pltpu.make_async_copy(hs_hbm.at[gidx(j)],
                                  rows_v.at[b], sem).wait()
            pltpu.sync_copy(rows_v.at[b], acc_sh.at[dst_v.at[j]], add=True)

            @pl.when(j + 2 < NCH)
            def _():
                pltpu.async_copy(hs_hbm.at[gidx(j + 2)], rows_v.at[b], sem)

        def body(t, c):
            for b in range(2):
                step(t * 2 + b, b)
            return c

        lax.fori_loop(0, NCH // 2, body, 0)
        if NCH % 2:
            step(NCH - 1, (NCH - 1) % 2)
        plsc.subcore_barrier()
        for r in range((NRC + NS - 1) // NS):
            chunk = r * NS + sid

            @pl.when(chunk < NRC)
            def _():
                base = pl.multiple_of(chunk * RST, 8)
                pltpu.sync_copy(acc_sh.at[pl.ds(base, RST)],
                                out_hbm.at[cid, pl.ds(base, RST)])

    PCH = 80
    NPC = N // PCH

    @functools.partial(
        pl.kernel,
        mesh=mesh,
        out_type=jax.ShapeDtypeStruct((NW, G, D), F32),
        scratch_types=[
            pltpu.VMEM((PCH, D), F32),
            pltpu.VMEM((PCH, D), F32),
            pltpu.VMEM((PCH, D), F32),
            pltpu.VMEM((PCH,), F32),
            pltpu.VMEM((PCH,), jnp.int32),
            pltpu.VMEM((D,), F32),
            pltpu.VMEM((G + 8, D), F32),
        ],
    )
    def _sc_pool(a_hbm, hs_hbm, dis_hbm, batch_hbm, b3_hbm, out_hbm,
                 a0_v, a1_v, hs_v, dis_v, bat_v, b3_v, out_v):
        cid = lax.axis_index("c")
        sid = lax.axis_index("s")
        wid = sid * NC + cid
        pltpu.sync_copy(b3_hbm, b3_v)

        def ibody(g, c):
            for k in range(8):
                out_v[g, pl.ds(k * 16, 16)] = jnp.full((16,), -jnp.inf, F32)
            return c

        lax.fori_loop(0, G + 8, ibody, 0)

        def do_chunk(chunk):
            base = pl.multiple_of(chunk * PCH, 8)
            pltpu.sync_copy(a_hbm.at[0, pl.ds(base, PCH)], a0_v)
            pltpu.sync_copy(a_hbm.at[1, pl.ds(base, PCH)], a1_v)
            pltpu.sync_copy(hs_hbm.at[pl.ds(base, PCH)], hs_v)
            pltpu.sync_copy(dis_hbm.at[pl.ds(base, PCH)], dis_v)
            pltpu.sync_copy(batch_hbm.at[pl.ds(base, PCH)], bat_v)

            def flush(prev, accs):
                for k in range(8):
                    sl = pl.ds(k * 16, 16)
                    out_v[prev, sl] = jnp.maximum(out_v[prev, sl], accs[k])

            def gbody(g, carry):
                prev = carry[0]
                accs = list(carry[1:])
                bat16 = bat_v[pl.ds(g * 16, 16)]
                dis16 = dis_v[pl.ds(g * 16, 16)]
                for l in range(16):
                    i = g * 16 + l
                    gid = bat16[l]
                    dv = dis16[l]
                    changed = gid != prev

                    @pl.when(changed)
                    def _():
                        flush(prev, accs)

                    new_accs = []
                    for k in range(8):
                        sl = pl.ds(k * 16, 16)
                        h = a0_v[i, sl] + a1_v[i, sl] - hs_v[i, sl]
                        h = jnp.maximum(dv * h + b3_v[sl], 0.0)
                        new_accs.append(
                            jnp.where(changed, h, jnp.maximum(accs[k], h)))
                    accs = new_accs
                    prev = gid
                return (prev, *accs)

            ninf = jnp.full((16,), -jnp.inf, F32)
            carry0 = (jnp.int32(G), ninf, ninf, ninf, ninf,
                      ninf, ninf, ninf, ninf)
            fin = lax.fori_loop(0, PCH // 16, gbody, carry0)
            flush(fin[0], list(fin[1:]))

        for t in range((NPC + NW - 1) // NW):
            chunk = t * NW + wid

            @pl.when(chunk < NPC)
            def _():
                do_chunk(chunk)

        pltpu.sync_copy(out_v.at[pl.ds(0, G)], out_hbm.at[wid])

    return _sc_deg, _sc_scatter, _sc_pool


def _p0_body(degp_ref, x_ref, w_ref, dis_ref, hs_ref):
    ones = jnp.ones((NW, 1), F32)
    deg = lax.dot_general(degp_ref[...], ones, (((0,), (0,)), ((), ())),
                          preferred_element_type=F32)
    dis = lax.rsqrt(deg + 1.0)
    dis_ref[...] = dis
    hs_ref[...] = dis * jnp.dot(x_ref[...], w_ref[...],
                                preferred_element_type=F32)


def _tc_p0(x, w1, degp):
    return pl.pallas_call(
        _p0_body,
        out_shape=[
            jax.ShapeDtypeStruct((N, 1), F32),
            jax.ShapeDtypeStruct((N, D), F32),
        ],
    )(degp, x, w1)


def _c_body(a_ref, hs_ref, dis_ref, b_ref, w_ref, out_ref):
    s = a_ref[0] + a_ref[1] - hs_ref[...]
    h = jnp.maximum(dis_ref[...] * s + b_ref[...], 0.0)
    out_ref[...] = dis_ref[...] * jnp.dot(h, w_ref[...],
                                          preferred_element_type=F32)


def _tc_combine(a, hs, dis, b, w_next):
    return pl.pallas_call(
        _c_body,
        grid=(GRID,),
        in_specs=[
            pl.BlockSpec((NC, BLK, D), lambda i: (0, i, 0)),
            pl.BlockSpec((BLK, D), lambda i: (i, 0)),
            pl.BlockSpec((BLK, 1), lambda i: (i, 0)),
            pl.BlockSpec((1, D), lambda i: (0, 0)),
            pl.BlockSpec((D, D), lambda i: (0, 0)),
        ],
        out_specs=pl.BlockSpec((BLK, D), lambda i: (i, 0)),
        out_shape=jax.ShapeDtypeStruct((N, D), F32),
    )(a, hs, dis, b, w_next)


def _head_body(p_ref, lw1_ref, lb1_ref, lw2_ref, lb2_ref, out_ref):
    gmax = jnp.max(p_ref[...], axis=0)
    g2 = jnp.dot(gmax, lw1_ref[...],
                 preferred_element_type=F32) + lb1_ref[...]
    out_ref[...] = jnp.dot(g2, lw2_ref[...],
                           preferred_element_type=F32) + lb2_ref[...]


def _tc_head(partials, lw1, lb1, lw2, lb2):
    return pl.pallas_call(
        _head_body,
        out_shape=jax.ShapeDtypeStruct((G, 1), F32),
    )(partials, lw1, lb1, lw2, lb2)


def kernel(x, x2, edge_attr, edge_attr2, edge_index, batch,
           W1, b1, W2, b2, W3, b3, LW1, Lb1, LW2, Lb2):
    src3 = edge_index[0].reshape(NW, EPW)
    dst3 = edge_index[1].reshape(NW, NCH, CH)
    dst16 = edge_index[1].reshape(NW, EPW // 16, 16)
    b1r = b1.reshape(1, D)
    b2r = b2.reshape(1, D)
    lb1r = Lb1.reshape(1, D)
    lb2r = Lb2.reshape(1, 1)

    _sc_deg, _sc_scatter, _sc_pool = _sc_kernels()
    degp = _sc_deg(dst16)
    dis, hs1 = _tc_p0(x, W1, degp)
    a1 = _sc_scatter(hs1, src3, dst3)
    hs2 = _tc_combine(a1, hs1, dis, b1r, W2)
    a2 = _sc_scatter(hs2, src3, dst3)
    hs3 = _tc_combine(a2, hs2, dis, b2r, W3)
    a3 = _sc_scatter(hs3, src3, dst3)
    partials = _sc_pool(a3, hs3, dis.reshape(N), batch, b3)
    out = _tc_head(partials, LW1, lb1r, LW2, lb2r)
    return out
